# Initial kernel scaffold; baseline (speedup 1.0000x reference)
import jax
import jax.numpy as jnp
from jax.experimental import pallas as pl


def kernel(embedding_weight, Acoeff, Bbasis, token_indices, j_indices):
    raise NotImplementedError("write your pallas kernel here")


# speedup vs baseline: 21.1824x; 21.1824x over previous
import dataclasses
import functools

import jax
import jax.numpy as jnp
from jax import lax
from jax.experimental import pallas as pl
from jax.experimental.pallas import tpu as pltpu
from jax.experimental.pallas import tpu_sc as plsc

M = 16
LT = 2500
NC = 2
NS = 16
NW = NC * NS
CHUNK = 512
GW = 128
SUB = CHUNK // GW


def _make_sc_kernel(n_tokens: int):
    per_w = n_tokens // NW
    steps = per_w // CHUNK
    mesh = plsc.VectorSubcoreMesh(core_axis_name="c", subcore_axis_name="s",
                                  num_cores=NC, num_subcores=NS)
    cp = pltpu.CompilerParams()
    fields = pltpu.CompilerParams.__dataclass_fields__
    if "needs_layout_passes" in fields:
        cp = dataclasses.replace(cp, needs_layout_passes=False)
    if "use_tc_tiling_on_sc" in fields:
        cp = dataclasses.replace(cp, use_tc_tiling_on_sc=False)

    @functools.partial(
        pl.kernel,
        out_type=jax.ShapeDtypeStruct((n_tokens, M), jnp.float32),
        mesh=mesh,
        compiler_params=cp,
        scratch_types=[
            pltpu.VMEM((LT, M), jnp.float32),
            pltpu.VMEM((M, LT), jnp.float32),
            pltpu.VMEM((CHUNK,), jnp.int32),
            pltpu.VMEM((CHUNK,), jnp.int32),
            pltpu.VMEM((CHUNK, M), jnp.float32),
            pltpu.VMEM((CHUNK, M), jnp.float32),
            pltpu.SemaphoreType.DMA,
        ],
    )
    def sc_kernel(emb_hbm, a_hbm, b_hbm, tok_hbm, j_hbm, out_hbm,
                  b_v, a_v, ti_v, ji_v, x_v, o_v, sem):
        wid = lax.axis_index("s") * NC + lax.axis_index("c")
        base = wid * per_w
        pltpu.sync_copy(b_hbm, b_v)
        pltpu.sync_copy(a_hbm, a_v)

        @pl.loop(0, steps)
        def _step(s):
            off = base + s * CHUNK
            pltpu.sync_copy(tok_hbm.at[pl.ds(off, CHUNK)], ti_v)
            pltpu.sync_copy(j_hbm.at[pl.ds(off, CHUNK)], ji_v)
            cps = [
                pltpu.async_copy(emb_hbm.at[ti_v.at[pl.ds(r * GW, GW)]],
                                 x_v.at[pl.ds(r * GW, GW)], sem)
                for r in range(SUB)
            ]
            for cp in cps:
                cp.wait()

            @pl.loop(0, CHUNK // 16)
            def _group(g):
                rows = lax.iota(jnp.int32, 16) + g * 16
                jv = ji_v[pl.ds(g * 16, 16)]
                accs = []
                for k in range(M):
                    kc = jnp.full((16,), k, jnp.int32)
                    xk = plsc.load_gather(x_v, [rows, kc])
                    bk = plsc.load_gather(b_v, [jv, kc])
                    p = xk * bk
                    if k < 4:
                        accs.append(p)
                    else:
                        accs[k % 4] = accs[k % 4] + p
                acc = (accs[0] + accs[1]) + (accs[2] + accs[3])
                for i in range(M):
                    ic = jnp.full((16,), i, jnp.int32)
                    ai = plsc.load_gather(a_v, [ic, jv])
                    plsc.store_scatter(o_v, [rows, ic], ai * acc)

            pltpu.sync_copy(o_v, out_hbm.at[pl.ds(off, CHUNK)])

    return sc_kernel


def kernel(embedding_weight, Acoeff, Bbasis, token_indices, j_indices):
    n = token_indices.shape[0]
    tok = token_indices.astype(jnp.int32)
    jid = j_indices.astype(jnp.int32)
    return _make_sc_kernel(n)(embedding_weight, Acoeff, Bbasis, tok, jid)


# speedup vs baseline: 40.0187x; 1.8892x over previous
import dataclasses
import functools

import jax
import jax.numpy as jnp
from jax import lax
from jax.experimental import pallas as pl
from jax.experimental.pallas import tpu as pltpu
from jax.experimental.pallas import tpu_sc as plsc

M = 16
LT = 2500
NC = 2
NS = 16
NW = NC * NS
CHUNK = 512
GW = 128
SUB = CHUNK // GW


def _make_sc_kernel(n_tokens: int):
    per_w = n_tokens // NW
    steps = per_w // CHUNK
    ntile = n_tokens // GW
    mesh = plsc.VectorSubcoreMesh(core_axis_name="c", subcore_axis_name="s",
                                  num_cores=NC, num_subcores=NS)
    cp = pltpu.CompilerParams()
    fields = pltpu.CompilerParams.__dataclass_fields__
    if "needs_layout_passes" in fields:
        cp = dataclasses.replace(cp, needs_layout_passes=False)
    if "use_tc_tiling_on_sc" in fields:
        cp = dataclasses.replace(cp, use_tc_tiling_on_sc=False)

    @functools.partial(
        pl.kernel,
        out_type=jax.ShapeDtypeStruct((n_tokens * M,), jnp.float32),
        mesh=mesh,
        compiler_params=cp,
        scratch_types=[
            pltpu.VMEM((M, LT), jnp.float32),
            pltpu.VMEM((M, LT), jnp.float32),
            pltpu.VMEM((CHUNK,), jnp.int32),
            pltpu.VMEM((CHUNK,), jnp.int32),
            pltpu.VMEM((CHUNK, M), jnp.float32),
            pltpu.VMEM((2 * SUB * 8 * GW,), jnp.float32),
            pltpu.SemaphoreType.DMA,
        ],
    )
    def sc_kernel(emb_hbm, a_hbm, bt_hbm, tok_hbm, j_hbm, out_hbm,
                  bt_v, a_v, ti_v, ji_v, x_v, o_v, sem):
        wid = lax.axis_index("s") * NC + lax.axis_index("c")
        base = wid * per_w
        pltpu.sync_copy(bt_hbm, bt_v)
        pltpu.sync_copy(a_hbm, a_v)

        @pl.loop(0, steps)
        def _step(s):
            off = base + s * CHUNK
            pltpu.sync_copy(tok_hbm.at[pl.ds(off, CHUNK)], ti_v)
            pltpu.sync_copy(j_hbm.at[pl.ds(off, CHUNK)], ji_v)
            cps = [
                pltpu.async_copy(emb_hbm.at[ti_v.at[pl.ds(r * GW, GW)]],
                                 x_v.at[pl.ds(r * GW, GW)], sem)
                for r in range(SUB)
            ]
            for cp_ in cps:
                cp_.wait()

            @pl.loop(0, CHUNK // 16)
            def _group(g):
                g16 = g * 16
                rows = lax.iota(jnp.int32, 16) + g16
                jv = ji_v[pl.ds(g16, 16)]
                accs = []
                for k in range(M):
                    kc = jnp.full((16,), k, jnp.int32)
                    xk = plsc.load_gather(x_v, [rows, kc])
                    bk = plsc.load_gather(bt_v, [kc, jv])
                    p = xk * bk
                    if k < 4:
                        accs.append(p)
                    else:
                        accs[k % 4] = accs[k % 4] + p
                acc = (accs[0] + accs[1]) + (accs[2] + accs[3])
                tn128 = (g16 // GW) * GW
                lo = g16 - tn128
                for i in range(M):
                    ic = jnp.full((16,), i, jnp.int32)
                    ai = plsc.load_gather(a_v, [ic, jv])
                    dst = ((i // 8) * SUB * 8 * GW + (i % 8) * GW
                           + tn128 * 8 + lo)
                    o_v[pl.ds(dst, 16)] = ai * acc

            for ti in range(2):
                pltpu.sync_copy(
                    o_v.at[pl.ds(ti * SUB * 8 * GW, SUB * 8 * GW)],
                    out_hbm.at[pl.ds((ti * ntile + off // GW) * 8 * GW,
                                     SUB * 8 * GW)])

    return sc_kernel


def kernel(embedding_weight, Acoeff, Bbasis, token_indices, j_indices):
    n = token_indices.shape[0]
    tok = token_indices.astype(jnp.int32)
    jid = j_indices.astype(jnp.int32)
    flat = _make_sc_kernel(n)(embedding_weight, Acoeff, Bbasis.T, tok, jid)
    out = flat.reshape(2, n // GW, 8, GW).transpose(1, 3, 0, 2)
    return out.reshape(n, M)


# speedup vs baseline: 52.1551x; 1.3033x over previous
import dataclasses
import functools

import jax
import jax.numpy as jnp
from jax import lax
from jax.experimental import pallas as pl
from jax.experimental.pallas import tpu as pltpu
from jax.experimental.pallas import tpu_sc as plsc

M = 16
LT = 2500
NC = 2
NS = 16
NW = NC * NS
CHUNK = 512
GW = 128
SUB = CHUNK // GW
OHALF = SUB * 8 * GW


def _make_sc_kernel(n_tokens: int):
    per_w = n_tokens // NW
    steps = per_w // CHUNK
    ntile = n_tokens // GW
    assert steps >= 4
    mesh = plsc.VectorSubcoreMesh(core_axis_name="c", subcore_axis_name="s",
                                  num_cores=NC, num_subcores=NS)
    cp = pltpu.CompilerParams()
    fields = pltpu.CompilerParams.__dataclass_fields__
    if "needs_layout_passes" in fields:
        cp = dataclasses.replace(cp, needs_layout_passes=False)
    if "use_tc_tiling_on_sc" in fields:
        cp = dataclasses.replace(cp, use_tc_tiling_on_sc=False)

    @functools.partial(
        pl.kernel,
        out_type=jax.ShapeDtypeStruct((n_tokens * M,), jnp.float32),
        mesh=mesh,
        compiler_params=cp,
        scratch_types=[
            pltpu.VMEM((M, LT), jnp.float32),
            pltpu.VMEM((M, LT), jnp.float32),
            pltpu.VMEM((2, CHUNK), jnp.int32),
            pltpu.VMEM((2, CHUNK), jnp.int32),
            pltpu.VMEM((2, CHUNK, M), jnp.float32),
            pltpu.VMEM((2, 2 * OHALF), jnp.float32),
            pltpu.SemaphoreType.DMA,
            pltpu.SemaphoreType.DMA,
            pltpu.SemaphoreType.DMA,
            pltpu.SemaphoreType.DMA,
        ],
    )
    def sc_kernel(emb_hbm, a_hbm, bt_hbm, tok_hbm, j_hbm, out_hbm,
                  bt_v, a_v, ti_v, ji_v, x_v, o_v, isem, gsem, os0, os1):
        osem = (os0, os1)
        wid = lax.axis_index("s") * NC + lax.axis_index("c")
        base = wid * per_w
        pltpu.sync_copy(bt_hbm, bt_v)
        pltpu.sync_copy(a_hbm, a_v)

        def idx_copies(s, b, make):
            off = base + s * CHUNK
            return [
                make(tok_hbm.at[pl.ds(off, CHUNK)], ti_v.at[b], isem),
                make(j_hbm.at[pl.ds(off, CHUNK)], ji_v.at[b], isem),
            ]

        def gather_copies(b, make):
            return [
                make(emb_hbm.at[ti_v.at[b].at[pl.ds(r * GW, GW)]],
                     x_v.at[b].at[pl.ds(r * GW, GW)], gsem)
                for r in range(SUB)
            ]

        def store_copies(s, b, make):
            off = base + s * CHUNK
            return [
                make(o_v.at[b].at[pl.ds(ti * OHALF, OHALF)],
                     out_hbm.at[pl.ds((ti * ntile + off // GW) * 8 * GW,
                                      OHALF)], osem[b])
                for ti in range(2)
            ]

        def fire(copies):
            del copies

        def wait(copies):
            for c in copies:
                c.wait()

        def compute(b):
            @pl.loop(0, CHUNK // 16)
            def _group(g):
                g16 = g * 16
                rows = lax.iota(jnp.int32, 16) + g16
                jv = ji_v.at[b][pl.ds(g16, 16)]
                accs = []
                for k in range(M):
                    kc = jnp.full((16,), k, jnp.int32)
                    xk = plsc.load_gather(x_v.at[b], [rows, kc])
                    bk = plsc.load_gather(bt_v, [kc, jv])
                    p = xk * bk
                    if k < 4:
                        accs.append(p)
                    else:
                        accs[k % 4] = accs[k % 4] + p
                acc = (accs[0] + accs[1]) + (accs[2] + accs[3])
                tn128 = (g16 // GW) * GW
                lo = g16 - tn128
                for i in range(M):
                    ic = jnp.full((16,), i, jnp.int32)
                    ai = plsc.load_gather(a_v, [ic, jv])
                    dst = (i // 8) * OHALF + (i % 8) * GW + tn128 * 8 + lo
                    o_v.at[b][pl.ds(dst, 16)] = ai * acc

        def do_chunk(s, b, fire_next_gather, fire_idx2, wait_store):
            wait(gather_copies(b, pltpu.make_async_copy))
            if fire_next_gather:
                wait(idx_copies(s + 1, 1 - b, pltpu.make_async_copy))
                fire(gather_copies(1 - b, pltpu.async_copy))
            if wait_store:
                wait(store_copies(s - 2, b, pltpu.make_async_copy))
            compute(b)
            fire(store_copies(s, b, pltpu.async_copy))
            if fire_idx2:
                fire(idx_copies(s + 2, b, pltpu.async_copy))

        wait(idx_copies(0, 0, pltpu.async_copy))
        fire(gather_copies(0, pltpu.async_copy))
        fire(idx_copies(1, 1, pltpu.async_copy))

        do_chunk(0, 0, True, True, False)
        do_chunk(1, 1, True, True, False)

        @pl.loop(2, steps - 2, step=2)
        def _pair(s):
            do_chunk(s, 0, True, True, True)
            do_chunk(s + 1, 1, True, True, True)

        do_chunk(steps - 2, 0, True, False, True)
        do_chunk(steps - 1, 1, False, False, True)

        wait(store_copies(steps - 2, 0, pltpu.make_async_copy))
        wait(store_copies(steps - 1, 1, pltpu.make_async_copy))

    return sc_kernel


def kernel(embedding_weight, Acoeff, Bbasis, token_indices, j_indices):
    n = token_indices.shape[0]
    tok = token_indices.astype(jnp.int32)
    jid = j_indices.astype(jnp.int32)
    flat = _make_sc_kernel(n)(embedding_weight, Acoeff, Bbasis.T, tok, jid)
    out = flat.reshape(2, n // GW, 8, GW).transpose(1, 3, 0, 2)
    return out.reshape(n, M)


# speedup vs baseline: 66.4260x; 1.2736x over previous
import dataclasses
import functools

import jax
import jax.numpy as jnp
from jax import lax
from jax.experimental import pallas as pl
from jax.experimental.pallas import tpu as pltpu
from jax.experimental.pallas import tpu_sc as plsc

M = 16
LT = 2500
NC = 2
NS = 16
NW = NC * NS
CHUNK = 512
GW = 128
SUB = CHUNK // GW
OHALF = SUB * 8 * GW


def _make_sc_kernel(n_tokens: int):
    per_w = n_tokens // NW
    steps = per_w // CHUNK
    ntile = n_tokens // GW
    assert steps >= 4
    mesh = plsc.VectorSubcoreMesh(core_axis_name="c", subcore_axis_name="s",
                                  num_cores=NC, num_subcores=NS)
    cp = pltpu.CompilerParams()
    fields = pltpu.CompilerParams.__dataclass_fields__
    if "needs_layout_passes" in fields:
        cp = dataclasses.replace(cp, needs_layout_passes=False)
    if "use_tc_tiling_on_sc" in fields:
        cp = dataclasses.replace(cp, use_tc_tiling_on_sc=False)

    @functools.partial(
        pl.kernel,
        out_type=jax.ShapeDtypeStruct((n_tokens * M,), jnp.float32),
        mesh=mesh,
        compiler_params=cp,
        scratch_types=[
            pltpu.VMEM((M, LT), jnp.float32),
            pltpu.VMEM((M, LT), jnp.float32),
            pltpu.VMEM((2, CHUNK), jnp.int32),
            pltpu.VMEM((2, CHUNK), jnp.int32),
            pltpu.VMEM((2, CHUNK, M), jnp.float32),
            pltpu.VMEM((2, 2 * OHALF), jnp.float32),
            pltpu.SemaphoreType.DMA,
            pltpu.SemaphoreType.DMA,
            pltpu.SemaphoreType.DMA,
            pltpu.SemaphoreType.DMA,
        ],
    )
    def sc_kernel(emb_hbm, a_hbm, bt_hbm, tok_hbm, j_hbm, out_hbm,
                  bt_v, a_v, ti_v, ji_v, x_v, o_v, isem, gsem, os0, os1):
        osem = (os0, os1)
        wid = lax.axis_index("s") * NC + lax.axis_index("c")
        base = wid * per_w
        pltpu.sync_copy(bt_hbm, bt_v)
        pltpu.sync_copy(a_hbm, a_v)

        def idx_copies(s, b, make):
            off = base + s * CHUNK
            return [
                make(tok_hbm.at[pl.ds(off, CHUNK)], ti_v.at[b], isem),
                make(j_hbm.at[pl.ds(off, CHUNK)], ji_v.at[b], isem),
            ]

        def gather_copies(b, make):
            return [
                make(emb_hbm.at[ti_v.at[b].at[pl.ds(r * GW, GW)]],
                     x_v.at[b].at[pl.ds(r * GW, GW)], gsem)
                for r in range(SUB)
            ]

        def store_copies(s, b, make):
            off = base + s * CHUNK
            return [
                make(o_v.at[b].at[pl.ds(ti * OHALF, OHALF)],
                     out_hbm.at[pl.ds((ti * ntile + off // GW) * 8 * GW,
                                      OHALF)], osem[b])
                for ti in range(2)
            ]

        def fire(copies):
            del copies

        def wait(copies):
            for c in copies:
                c.wait()

        def compute(b):
            @plsc.parallel_loop(0, CHUNK // 16, 1)
            def _group(g):
                g16 = g * 16
                rows = lax.iota(jnp.int32, 16) + g16
                jv = ji_v.at[b][pl.ds(g16, 16)]
                accs = []
                for k in range(M):
                    kc = jnp.full((16,), k, jnp.int32)
                    xk = plsc.load_gather(x_v.at[b], [rows, kc])
                    bk = plsc.load_gather(bt_v, [kc, jv])
                    p = xk * bk
                    if k < 4:
                        accs.append(p)
                    else:
                        accs[k % 4] = accs[k % 4] + p
                acc = (accs[0] + accs[1]) + (accs[2] + accs[3])
                tn128 = (g16 // GW) * GW
                lo = g16 - tn128
                for i in range(M):
                    ic = jnp.full((16,), i, jnp.int32)
                    ai = plsc.load_gather(a_v, [ic, jv])
                    dst = (i // 8) * OHALF + (i % 8) * GW + tn128 * 8 + lo
                    o_v.at[b][pl.ds(dst, 16)] = ai * acc

        def do_chunk(s, b, fire_next_gather, fire_idx2, wait_store):
            wait(gather_copies(b, pltpu.make_async_copy))
            if fire_next_gather:
                wait(idx_copies(s + 1, 1 - b, pltpu.make_async_copy))
                fire(gather_copies(1 - b, pltpu.async_copy))
            if wait_store:
                wait(store_copies(s - 2, b, pltpu.make_async_copy))
            compute(b)
            fire(store_copies(s, b, pltpu.async_copy))
            if fire_idx2:
                fire(idx_copies(s + 2, b, pltpu.async_copy))

        wait(idx_copies(0, 0, pltpu.async_copy))
        fire(gather_copies(0, pltpu.async_copy))
        fire(idx_copies(1, 1, pltpu.async_copy))

        do_chunk(0, 0, True, True, False)
        do_chunk(1, 1, True, True, False)

        @pl.loop(2, steps - 2, step=2)
        def _pair(s):
            do_chunk(s, 0, True, True, True)
            do_chunk(s + 1, 1, True, True, True)

        do_chunk(steps - 2, 0, True, False, True)
        do_chunk(steps - 1, 1, False, False, True)

        wait(store_copies(steps - 2, 0, pltpu.make_async_copy))
        wait(store_copies(steps - 1, 1, pltpu.make_async_copy))

    return sc_kernel


def kernel(embedding_weight, Acoeff, Bbasis, token_indices, j_indices):
    n = token_indices.shape[0]
    tok = token_indices.astype(jnp.int32)
    jid = j_indices.astype(jnp.int32)
    flat = _make_sc_kernel(n)(embedding_weight, Acoeff, Bbasis.T, tok, jid)
    out = flat.reshape(2, n // GW, 8, GW).transpose(1, 3, 0, 2)
    return out.reshape(n, M)


# speedup vs baseline: 83.5333x; 1.2575x over previous
import dataclasses
import functools

import jax
import jax.numpy as jnp
from jax import lax
from jax.experimental import pallas as pl
from jax.experimental.pallas import tpu as pltpu
from jax.experimental.pallas import tpu_sc as plsc

M = 16
LT = 2500
NC = 2
NS = 16
NW = NC * NS
CHUNK = 512
GW = 128
SUB = CHUNK // GW
OHALF = SUB * 8 * GW


def _make_sc_kernel(n_tokens: int):
    per_w = n_tokens // NW
    steps = per_w // CHUNK
    ntile = n_tokens // GW
    assert steps >= 4
    mesh = plsc.VectorSubcoreMesh(core_axis_name="c", subcore_axis_name="s",
                                  num_cores=NC, num_subcores=NS)
    cp = pltpu.CompilerParams()
    fields = pltpu.CompilerParams.__dataclass_fields__
    if "needs_layout_passes" in fields:
        cp = dataclasses.replace(cp, needs_layout_passes=False)
    if "use_tc_tiling_on_sc" in fields:
        cp = dataclasses.replace(cp, use_tc_tiling_on_sc=False)

    @functools.partial(
        pl.kernel,
        out_type=jax.ShapeDtypeStruct((n_tokens * M,), jnp.float32),
        mesh=mesh,
        compiler_params=cp,
        scratch_types=[
            pltpu.VMEM((M * LT,), jnp.float32),
            pltpu.VMEM((2, CHUNK), jnp.int32),
            pltpu.VMEM((2, CHUNK), jnp.int32),
            pltpu.VMEM((2, CHUNK, M), jnp.float32),
            pltpu.VMEM((2, 2 * OHALF), jnp.float32),
            pltpu.SemaphoreType.DMA,
            pltpu.SemaphoreType.DMA,
            pltpu.SemaphoreType.DMA,
            pltpu.SemaphoreType.DMA,
        ],
    )
    def sc_kernel(emb_hbm, ct_hbm, tok_hbm, j_hbm, out_hbm,
                  ct_v, ti_v, ji_v, x_v, o_v, isem, gsem, os0, os1):
        osem = (os0, os1)
        wid = lax.axis_index("s") * NC + lax.axis_index("c")
        base = wid * per_w
        pltpu.sync_copy(ct_hbm, ct_v)

        def idx_copies(s, b, make):
            off = base + s * CHUNK
            return [
                make(tok_hbm.at[pl.ds(off, CHUNK)], ti_v.at[b], isem),
                make(j_hbm.at[pl.ds(off, CHUNK)], ji_v.at[b], isem),
            ]

        def gather_copies(b, make):
            return [
                make(emb_hbm.at[ti_v.at[b].at[pl.ds(r * GW, GW)]],
                     x_v.at[b].at[pl.ds(r * GW, GW)], gsem)
                for r in range(SUB)
            ]

        def store_copies(s, b, make):
            off = base + s * CHUNK
            return [
                make(o_v.at[b].at[pl.ds(ti * OHALF, OHALF)],
                     out_hbm.at[pl.ds((ti * ntile + off // GW) * 8 * GW,
                                      OHALF)], osem[b])
                for ti in range(2)
            ]

        def fire(copies):
            del copies

        def wait(copies):
            for c in copies:
                c.wait()

        def compute(b):
            @plsc.parallel_loop(0, CHUNK // 16, 1)
            def _group(g):
                g16 = g * 16
                rows = lax.iota(jnp.int32, 16) + g16
                jv = ji_v.at[b][pl.ds(g16, 16)]
                accs = []
                a_regs = []
                for k in range(M):
                    ct = plsc.load_gather(ct_v, [jv + (k * LT)])
                    bk, ak = plsc.unpack(
                        plsc.bitcast(ct, jnp.bfloat16),
                        format=plsc.PackFormat.INTERLEAVED,
                        preferred_element_type=jnp.float32)
                    a_regs.append(ak)
                    kc = jnp.full((16,), k, jnp.int32)
                    xk = plsc.load_gather(x_v.at[b], [rows, kc])
                    p = xk * bk
                    if k < 4:
                        accs.append(p)
                    else:
                        accs[k % 4] = accs[k % 4] + p
                acc = (accs[0] + accs[1]) + (accs[2] + accs[3])
                tn128 = (g16 // GW) * GW
                lo = g16 - tn128
                for i in range(M):
                    dst = (i // 8) * OHALF + (i % 8) * GW + tn128 * 8 + lo
                    o_v.at[b][pl.ds(dst, 16)] = a_regs[i] * acc

        def do_chunk(s, b, fire_next_gather, fire_idx2, wait_store):
            wait(gather_copies(b, pltpu.make_async_copy))
            if fire_next_gather:
                wait(idx_copies(s + 1, 1 - b, pltpu.make_async_copy))
                fire(gather_copies(1 - b, pltpu.async_copy))
            if wait_store:
                wait(store_copies(s - 2, b, pltpu.make_async_copy))
            compute(b)
            fire(store_copies(s, b, pltpu.async_copy))
            if fire_idx2:
                fire(idx_copies(s + 2, b, pltpu.async_copy))

        wait(idx_copies(0, 0, pltpu.async_copy))
        fire(gather_copies(0, pltpu.async_copy))
        fire(idx_copies(1, 1, pltpu.async_copy))

        do_chunk(0, 0, True, True, False)
        do_chunk(1, 1, True, True, False)

        @pl.loop(2, steps - 2, step=2)
        def _pair(s):
            do_chunk(s, 0, True, True, True)
            do_chunk(s + 1, 1, True, True, True)

        do_chunk(steps - 2, 0, True, False, True)
        do_chunk(steps - 1, 1, False, False, True)

        wait(store_copies(steps - 2, 0, pltpu.make_async_copy))
        wait(store_copies(steps - 1, 1, pltpu.make_async_copy))

    return sc_kernel


def kernel(embedding_weight, Acoeff, Bbasis, token_indices, j_indices):
    n = token_indices.shape[0]
    tok = token_indices.astype(jnp.int32)
    jid = j_indices.astype(jnp.int32)
    bu = jax.lax.bitcast_convert_type(
        Bbasis.T.astype(jnp.bfloat16), jnp.uint16).astype(jnp.uint32)
    au = jax.lax.bitcast_convert_type(
        Acoeff.astype(jnp.bfloat16), jnp.uint16).astype(jnp.uint32)
    ct = jax.lax.bitcast_convert_type(bu | (au << 16),
                                      jnp.float32).reshape(-1)
    flat = _make_sc_kernel(n)(embedding_weight, ct, tok, jid)
    out = flat.reshape(2, n // GW, 8, GW).transpose(1, 3, 0, 2)
    return out.reshape(n, M)


# speedup vs baseline: 84.5062x; 1.0116x over previous
import dataclasses
import functools

import jax
import jax.numpy as jnp
from jax import lax
from jax.experimental import pallas as pl
from jax.experimental.pallas import tpu as pltpu
from jax.experimental.pallas import tpu_sc as plsc

M = 16
LT = 2500
NC = 2
NS = 16
NW = NC * NS
CHUNK = 512
GW = 128
SUB = CHUNK // GW
OHALF = SUB * 8 * GW


def _make_sc_kernel(n_tokens: int):
    per_w = n_tokens // NW
    steps = per_w // CHUNK
    ntile = n_tokens // GW
    assert steps >= 4
    mesh = plsc.VectorSubcoreMesh(core_axis_name="c", subcore_axis_name="s",
                                  num_cores=NC, num_subcores=NS)
    cp = pltpu.CompilerParams()
    fields = pltpu.CompilerParams.__dataclass_fields__
    if "needs_layout_passes" in fields:
        cp = dataclasses.replace(cp, needs_layout_passes=False)
    if "use_tc_tiling_on_sc" in fields:
        cp = dataclasses.replace(cp, use_tc_tiling_on_sc=False)

    @functools.partial(
        pl.kernel,
        out_type=jax.ShapeDtypeStruct((n_tokens * M,), jnp.float32),
        mesh=mesh,
        compiler_params=cp,
        scratch_types=[
            pltpu.VMEM((M * LT,), jnp.float32),
            pltpu.VMEM((2, CHUNK), jnp.int32),
            pltpu.VMEM((2, CHUNK), jnp.int32),
            pltpu.VMEM((2, CHUNK, M), jnp.float32),
            pltpu.VMEM((2, 2 * OHALF), jnp.float32),
            pltpu.SemaphoreType.DMA,
            pltpu.SemaphoreType.DMA,
            pltpu.SemaphoreType.DMA,
            pltpu.SemaphoreType.DMA,
        ],
    )
    def sc_kernel(emb_hbm, ct_hbm, tok_hbm, j_hbm, out_hbm,
                  ct_v, ti_v, ji_v, x_v, o_v, isem, gsem, os0, os1):
        osem = (os0, os1)
        wid = lax.axis_index("s") * NC + lax.axis_index("c")
        base = wid * per_w
        pltpu.sync_copy(ct_hbm, ct_v)

        def idx_copies(s, b, make):
            off = base + s * CHUNK
            return [
                make(tok_hbm.at[pl.ds(off, CHUNK)], ti_v.at[b], isem),
                make(j_hbm.at[pl.ds(off, CHUNK)], ji_v.at[b], isem),
            ]

        def gather_copies(b, make):
            return [
                make(emb_hbm.at[ti_v.at[b].at[pl.ds(r * GW, GW)]],
                     x_v.at[b].at[pl.ds(r * GW, GW)], gsem)
                for r in range(SUB)
            ]

        def store_copies(s, b, make):
            off = base + s * CHUNK
            return [
                make(o_v.at[b].at[pl.ds(ti * OHALF, OHALF)],
                     out_hbm.at[pl.ds((ti * ntile + off // GW) * 8 * GW,
                                      OHALF)], osem[b])
                for ti in range(2)
            ]

        def fire(copies):
            del copies

        def wait(copies):
            for c in copies:
                c.wait()

        def compute(b):
            @plsc.parallel_loop(0, CHUNK // 16, 1, unroll=2)
            def _group(g):
                g16 = g * 16
                rows = lax.iota(jnp.int32, 16) + g16
                jv = ji_v.at[b][pl.ds(g16, 16)]
                accs = []
                a_regs = []
                for k in range(M):
                    ct = plsc.load_gather(ct_v, [jv + (k * LT)])
                    bk, ak = plsc.unpack(
                        plsc.bitcast(ct, jnp.bfloat16),
                        format=plsc.PackFormat.INTERLEAVED,
                        preferred_element_type=jnp.float32)
                    a_regs.append(ak)
                    kc = jnp.full((16,), k, jnp.int32)
                    xk = plsc.load_gather(x_v.at[b], [rows, kc])
                    p = xk * bk
                    if k < 4:
                        accs.append(p)
                    else:
                        accs[k % 4] = accs[k % 4] + p
                acc = (accs[0] + accs[1]) + (accs[2] + accs[3])
                tn128 = (g16 // GW) * GW
                lo = g16 - tn128
                for i in range(M):
                    dst = (i // 8) * OHALF + (i % 8) * GW + tn128 * 8 + lo
                    o_v.at[b][pl.ds(dst, 16)] = a_regs[i] * acc

        def do_chunk(s, b, fire_next_gather, fire_idx2, wait_store):
            wait(gather_copies(b, pltpu.make_async_copy))
            if fire_next_gather:
                wait(idx_copies(s + 1, 1 - b, pltpu.make_async_copy))
                fire(gather_copies(1 - b, pltpu.async_copy))
            if wait_store:
                wait(store_copies(s - 2, b, pltpu.make_async_copy))
            compute(b)
            fire(store_copies(s, b, pltpu.async_copy))
            if fire_idx2:
                fire(idx_copies(s + 2, b, pltpu.async_copy))

        wait(idx_copies(0, 0, pltpu.async_copy))
        fire(gather_copies(0, pltpu.async_copy))
        fire(idx_copies(1, 1, pltpu.async_copy))

        do_chunk(0, 0, True, True, False)
        do_chunk(1, 1, True, True, False)

        @pl.loop(2, steps - 2, step=2)
        def _pair(s):
            do_chunk(s, 0, True, True, True)
            do_chunk(s + 1, 1, True, True, True)

        do_chunk(steps - 2, 0, True, False, True)
        do_chunk(steps - 1, 1, False, False, True)

        wait(store_copies(steps - 2, 0, pltpu.make_async_copy))
        wait(store_copies(steps - 1, 1, pltpu.make_async_copy))

    return sc_kernel


def kernel(embedding_weight, Acoeff, Bbasis, token_indices, j_indices):
    n = token_indices.shape[0]
    tok = token_indices.astype(jnp.int32)
    jid = j_indices.astype(jnp.int32)
    bu = jax.lax.bitcast_convert_type(
        Bbasis.T.astype(jnp.bfloat16), jnp.uint16).astype(jnp.uint32)
    au = jax.lax.bitcast_convert_type(
        Acoeff.astype(jnp.bfloat16), jnp.uint16).astype(jnp.uint32)
    ct = jax.lax.bitcast_convert_type(bu | (au << 16),
                                      jnp.float32).reshape(-1)
    flat = _make_sc_kernel(n)(embedding_weight, ct, tok, jid)
    out = flat.reshape(2, n // GW, 8, GW).transpose(1, 3, 0, 2)
    return out.reshape(n, M)


# speedup vs baseline: 90.7229x; 1.0736x over previous
import dataclasses
import functools

import jax
import jax.numpy as jnp
from jax import lax
from jax.experimental import pallas as pl
from jax.experimental.pallas import tpu as pltpu
from jax.experimental.pallas import tpu_sc as plsc

M = 16
LT = 2500
NC = 2
NS = 16
NW = NC * NS
CHUNK = 1024
GW = 128
SUB = CHUNK // GW
OHALF = SUB * 8 * GW


def _make_sc_kernel(n_tokens: int):
    per_w = n_tokens // NW
    steps = per_w // CHUNK
    ntile = n_tokens // GW
    assert steps >= 4
    mesh = plsc.VectorSubcoreMesh(core_axis_name="c", subcore_axis_name="s",
                                  num_cores=NC, num_subcores=NS)
    cp = pltpu.CompilerParams()
    fields = pltpu.CompilerParams.__dataclass_fields__
    if "needs_layout_passes" in fields:
        cp = dataclasses.replace(cp, needs_layout_passes=False)
    if "use_tc_tiling_on_sc" in fields:
        cp = dataclasses.replace(cp, use_tc_tiling_on_sc=False)

    @functools.partial(
        pl.kernel,
        out_type=jax.ShapeDtypeStruct((n_tokens * M,), jnp.float32),
        mesh=mesh,
        compiler_params=cp,
        scratch_types=[
            pltpu.VMEM((M * LT,), jnp.float32),
            pltpu.VMEM((2, CHUNK), jnp.int32),
            pltpu.VMEM((2, CHUNK), jnp.int32),
            pltpu.VMEM((2, CHUNK, M), jnp.float32),
            pltpu.VMEM((2, 2 * OHALF), jnp.float32),
            pltpu.SemaphoreType.DMA,
            pltpu.SemaphoreType.DMA,
            pltpu.SemaphoreType.DMA,
            pltpu.SemaphoreType.DMA,
        ],
    )
    def sc_kernel(emb_hbm, ct_hbm, tok_hbm, j_hbm, out_hbm,
                  ct_v, ti_v, ji_v, x_v, o_v, isem, gsem, os0, os1):
        osem = (os0, os1)
        wid = lax.axis_index("s") * NC + lax.axis_index("c")
        base = wid * per_w
        pltpu.sync_copy(ct_hbm, ct_v)

        def idx_copies(s, b, make):
            off = base + s * CHUNK
            return [
                make(tok_hbm.at[pl.ds(off, CHUNK)], ti_v.at[b], isem),
                make(j_hbm.at[pl.ds(off, CHUNK)], ji_v.at[b], isem),
            ]

        def gather_copies(b, make):
            return [
                make(emb_hbm.at[ti_v.at[b].at[pl.ds(r * GW, GW)]],
                     x_v.at[b].at[pl.ds(r * GW, GW)], gsem)
                for r in range(SUB)
            ]

        def store_copies(s, b, make):
            off = base + s * CHUNK
            return [
                make(o_v.at[b].at[pl.ds(ti * OHALF, OHALF)],
                     out_hbm.at[pl.ds((ti * ntile + off // GW) * 8 * GW,
                                      OHALF)], osem[b])
                for ti in range(2)
            ]

        def fire(copies):
            del copies

        def wait(copies):
            for c in copies:
                c.wait()

        def compute(b):
            @plsc.parallel_loop(0, CHUNK // 16, 1, unroll=2)
            def _group(g):
                g16 = g * 16
                rows = lax.iota(jnp.int32, 16) + g16
                jv = ji_v.at[b][pl.ds(g16, 16)]
                accs = []
                a_regs = []
                for k in range(M):
                    ct = plsc.load_gather(ct_v, [jv + (k * LT)])
                    bk, ak = plsc.unpack(
                        plsc.bitcast(ct, jnp.bfloat16),
                        format=plsc.PackFormat.INTERLEAVED,
                        preferred_element_type=jnp.float32)
                    a_regs.append(ak)
                    kc = jnp.full((16,), k, jnp.int32)
                    xk = plsc.load_gather(x_v.at[b], [rows, kc])
                    p = xk * bk
                    if k < 4:
                        accs.append(p)
                    else:
                        accs[k % 4] = accs[k % 4] + p
                acc = (accs[0] + accs[1]) + (accs[2] + accs[3])
                tn128 = (g16 // GW) * GW
                lo = g16 - tn128
                for i in range(M):
                    dst = (i // 8) * OHALF + (i % 8) * GW + tn128 * 8 + lo
                    o_v.at[b][pl.ds(dst, 16)] = a_regs[i] * acc

        def do_chunk(s, b, fire_next_gather, fire_idx2, wait_store):
            wait(gather_copies(b, pltpu.make_async_copy))
            if fire_next_gather:
                wait(idx_copies(s + 1, 1 - b, pltpu.make_async_copy))
                fire(gather_copies(1 - b, pltpu.async_copy))
            if wait_store:
                wait(store_copies(s - 2, b, pltpu.make_async_copy))
            compute(b)
            fire(store_copies(s, b, pltpu.async_copy))
            if fire_idx2:
                fire(idx_copies(s + 2, b, pltpu.async_copy))

        wait(idx_copies(0, 0, pltpu.async_copy))
        fire(gather_copies(0, pltpu.async_copy))
        fire(idx_copies(1, 1, pltpu.async_copy))

        do_chunk(0, 0, True, True, False)
        do_chunk(1, 1, True, True, False)

        @pl.loop(2, steps - 2, step=2)
        def _pair(s):
            do_chunk(s, 0, True, True, True)
            do_chunk(s + 1, 1, True, True, True)

        do_chunk(steps - 2, 0, True, False, True)
        do_chunk(steps - 1, 1, False, False, True)

        wait(store_copies(steps - 2, 0, pltpu.make_async_copy))
        wait(store_copies(steps - 1, 1, pltpu.make_async_copy))

    return sc_kernel


def kernel(embedding_weight, Acoeff, Bbasis, token_indices, j_indices):
    n = token_indices.shape[0]
    tok = token_indices.astype(jnp.int32)
    jid = j_indices.astype(jnp.int32)
    bu = jax.lax.bitcast_convert_type(
        Bbasis.T.astype(jnp.bfloat16), jnp.uint16).astype(jnp.uint32)
    au = jax.lax.bitcast_convert_type(
        Acoeff.astype(jnp.bfloat16), jnp.uint16).astype(jnp.uint32)
    ct = jax.lax.bitcast_convert_type(bu | (au << 16),
                                      jnp.float32).reshape(-1)
    flat = _make_sc_kernel(n)(embedding_weight, ct, tok, jid)
    out = flat.reshape(2, n // GW, 8, GW).transpose(1, 3, 0, 2)
    return out.reshape(n, M)


# speedup vs baseline: 119.1392x; 1.3132x over previous
import dataclasses
import functools

import jax
import jax.numpy as jnp
from jax import lax
from jax.experimental import pallas as pl
from jax.experimental.pallas import tpu as pltpu
from jax.experimental.pallas import tpu_sc as plsc

M = 16
LT = 2500
NC = 2
NS = 16
NW = NC * NS
CHUNK = 1024
GW = 128
SUB = CHUNK // GW
OHALF = SUB * 8 * GW


def _make_sc_kernel(n_tokens: int):
    per_w = n_tokens // NW
    steps = per_w // CHUNK
    ntile = n_tokens // GW
    assert steps >= 4
    mesh = plsc.VectorSubcoreMesh(core_axis_name="c", subcore_axis_name="s",
                                  num_cores=NC, num_subcores=NS)
    cp = pltpu.CompilerParams()
    fields = pltpu.CompilerParams.__dataclass_fields__
    if "needs_layout_passes" in fields:
        cp = dataclasses.replace(cp, needs_layout_passes=False)
    if "use_tc_tiling_on_sc" in fields:
        cp = dataclasses.replace(cp, use_tc_tiling_on_sc=False)

    @functools.partial(
        pl.kernel,
        out_type=jax.ShapeDtypeStruct((n_tokens * M,), jnp.float32),
        mesh=mesh,
        compiler_params=cp,
        scratch_types=[
            pltpu.VMEM((M * LT,), jnp.float32),
            pltpu.VMEM((2, CHUNK), jnp.int32),
            pltpu.VMEM((2, CHUNK), jnp.int32),
            pltpu.VMEM((2, CHUNK, M), jnp.float32),
            pltpu.VMEM((2, 2 * OHALF), jnp.float32),
            pltpu.SemaphoreType.DMA,
            pltpu.SemaphoreType.DMA,
            pltpu.SemaphoreType.DMA,
            pltpu.SemaphoreType.DMA,
        ],
    )
    def sc_kernel(emb_hbm, ct_hbm, tok_hbm, j_hbm, out_hbm,
                  ct_v, ti_v, ji_v, x_v, o_v, isem, gsem, os0, os1):
        osem = (os0, os1)
        wid = lax.axis_index("s") * NC + lax.axis_index("c")
        base = wid * per_w
        pltpu.sync_copy(ct_hbm, ct_v)

        def idx_copies(s, b, make):
            off = base + s * CHUNK
            return [
                make(tok_hbm.at[pl.ds(off, CHUNK)], ti_v.at[b], isem),
                make(j_hbm.at[pl.ds(off, CHUNK)], ji_v.at[b], isem),
            ]

        def gather_copies(b, make):
            return [
                make(emb_hbm.at[ti_v.at[b].at[pl.ds(r * GW, GW)]],
                     x_v.at[b].at[pl.ds(r * GW, GW)], gsem)
                for r in range(SUB)
            ]

        def store_copies(s, b, make):
            off = base + s * CHUNK
            return [
                make(o_v.at[b].at[pl.ds(ti * OHALF, OHALF)],
                     out_hbm.at[pl.ds((ti * ntile + off // GW) * 8 * GW,
                                      OHALF)], osem[b])
                for ti in range(2)
            ]

        def fire(copies):
            del copies

        def wait(copies):
            for c in copies:
                c.wait()

        def compute(b):
            @plsc.parallel_loop(0, CHUNK // 16, 1, unroll=2)
            def _group(g):
                g16 = g * 16
                rows = lax.iota(jnp.int32, 16) + g16
                jv = ji_v.at[b][pl.ds(g16, 16)]
                accs = []
                a_regs = []
                for k in range(M):
                    ct = plsc.load_gather(ct_v, [jv + (k * LT)])
                    bk, ak = plsc.unpack(
                        plsc.bitcast(ct, jnp.bfloat16),
                        format=plsc.PackFormat.INTERLEAVED,
                        preferred_element_type=jnp.float32)
                    a_regs.append(ak)
                    kc = jnp.full((16,), k, jnp.int32)
                    xk = plsc.load_gather(x_v.at[b], [rows, kc])
                    p = xk * bk
                    if k < 4:
                        accs.append(p)
                    else:
                        accs[k % 4] = accs[k % 4] + p
                acc = (accs[0] + accs[1]) + (accs[2] + accs[3])
                tn128 = (g16 // GW) * GW
                lo = g16 - tn128
                for i in range(M):
                    dst = (i // 8) * OHALF + (i % 8) * GW + tn128 * 8 + lo
                    o_v.at[b][pl.ds(dst, 16)] = a_regs[i] * acc

        def do_chunk(s, b, fire_next_gather, fire_idx2, wait_store):
            wait(gather_copies(b, pltpu.make_async_copy))
            if fire_next_gather:
                wait(idx_copies(s + 1, 1 - b, pltpu.make_async_copy))
                fire(gather_copies(1 - b, pltpu.async_copy))
            if wait_store:
                wait(store_copies(s - 2, b, pltpu.make_async_copy))
            compute(b)
            fire(store_copies(s, b, pltpu.async_copy))
            if fire_idx2:
                fire(idx_copies(s + 2, b, pltpu.async_copy))

        wait(idx_copies(0, 0, pltpu.async_copy))
        fire(gather_copies(0, pltpu.async_copy))
        fire(idx_copies(1, 1, pltpu.async_copy))

        do_chunk(0, 0, True, True, False)
        do_chunk(1, 1, True, True, False)

        @pl.loop(2, steps - 2, step=2)
        def _pair(s):
            do_chunk(s, 0, True, True, True)
            do_chunk(s + 1, 1, True, True, True)

        do_chunk(steps - 2, 0, True, False, True)
        do_chunk(steps - 1, 1, False, False, True)

        wait(store_copies(steps - 2, 0, pltpu.make_async_copy))
        wait(store_copies(steps - 1, 1, pltpu.make_async_copy))

    return sc_kernel


def _make_fmt_kernel(vocab: int):
    vb = vocab // GW
    bps = 4
    half = vb * 8
    steps = vb // (NW * bps)
    mesh = plsc.VectorSubcoreMesh(core_axis_name="c", subcore_axis_name="s",
                                  num_cores=NC, num_subcores=NS)
    cp = pltpu.CompilerParams()
    fields = pltpu.CompilerParams.__dataclass_fields__
    if "needs_layout_passes" in fields:
        cp = dataclasses.replace(cp, needs_layout_passes=False)
    if "use_tc_tiling_on_sc" in fields:
        cp = dataclasses.replace(cp, use_tc_tiling_on_sc=False)

    @functools.partial(
        pl.kernel,
        out_type=jax.ShapeDtypeStruct((vocab, M), jnp.float32),
        mesh=mesh,
        compiler_params=cp,
        scratch_types=[
            pltpu.VMEM((2, 2, 8 * bps, GW), jnp.float32),
            pltpu.VMEM((2, GW * bps, M), jnp.float32),
            pltpu.SemaphoreType.DMA,
            pltpu.SemaphoreType.DMA,
            pltpu.SemaphoreType.DMA,
        ],
    )
    def fmt_kernel(e4_hbm, out_hbm, in_v, out_v, isem, os0, os1):
        osem = (os0, os1)
        wid = lax.axis_index("s") * NC + lax.axis_index("c")
        base = wid * steps * bps

        def in_copies(s, b, make):
            tn0 = base + s * bps
            return [
                make(e4_hbm.at[pl.ds(ti * half + tn0 * 8, 8 * bps)],
                     in_v.at[b].at[ti], isem)
                for ti in range(2)
            ]

        def out_copies(s, b, make):
            tn0 = base + s * bps
            return [make(out_v.at[b],
                         out_hbm.at[pl.ds(tn0 * GW, GW * bps)], osem[b])]

        def wait(copies):
            for c in copies:
                c.wait()

        def transpose_block(b):
            lane = lax.iota(jnp.int32, 16)
            for blk in range(bps):
                @pl.loop(0, 8)
                def _g(g):
                    l0 = g * 16
                    toks = lane + (blk * GW + l0)
                    for f in range(M):
                        v = in_v.at[b][f // 8, blk * 8 + (f % 8),
                                       pl.ds(l0, 16)]
                        plsc.store_scatter(
                            out_v.at[b], [toks, jnp.full((16,), f, jnp.int32)],
                            v)

        def do_step(s, b, fire_in_next, wait_out):
            wait(in_copies(s, b, pltpu.make_async_copy))
            if fire_in_next:
                in_copies(s + 1, 1 - b, pltpu.async_copy)
            if wait_out:
                wait(out_copies(s - 2, b, pltpu.make_async_copy))
            transpose_block(b)
            out_copies(s, b, pltpu.async_copy)

        in_copies(0, 0, pltpu.async_copy)
        do_step(0, 0, True, False)
        do_step(1, 1, True, False)

        @pl.loop(2, steps - 2, step=2)
        def _pair(s):
            do_step(s, 0, True, True)
            do_step(s + 1, 1, True, True)

        do_step(steps - 2, 0, True, True)
        do_step(steps - 1, 1, False, True)
        wait(out_copies(steps - 2, 0, pltpu.make_async_copy))
        wait(out_copies(steps - 1, 1, pltpu.make_async_copy))

    return fmt_kernel


def kernel(embedding_weight, Acoeff, Bbasis, token_indices, j_indices):
    n = token_indices.shape[0]
    tok = token_indices.astype(jnp.int32)
    jid = j_indices.astype(jnp.int32)
    bu = jax.lax.bitcast_convert_type(
        Bbasis.T.astype(jnp.bfloat16), jnp.uint16).astype(jnp.uint32)
    au = jax.lax.bitcast_convert_type(
        Acoeff.astype(jnp.bfloat16), jnp.uint16).astype(jnp.uint32)
    ct = jax.lax.bitcast_convert_type(bu | (au << 16),
                                      jnp.float32).reshape(-1)
    vocab = embedding_weight.shape[0]
    vb = vocab // GW
    e4 = (embedding_weight.T.reshape(2, 8, vb, GW)
          .transpose(0, 2, 1, 3).reshape(2 * vb * 8, GW))
    emb_lin = _make_fmt_kernel(vocab)(e4)
    flat = _make_sc_kernel(n)(emb_lin, ct, tok, jid)
    out = flat.reshape(2, n // GW, 8, GW).transpose(1, 3, 0, 2)
    return out.reshape(n, M)
